# speedup vs baseline: 6.3253x; 6.3253x over previous
import functools

import jax
import jax.numpy as jnp
from jax import lax
from jax.experimental import pallas as pl
from jax.experimental.pallas import tpu as pltpu
from jax.experimental.pallas import tpu_sc as plsc

D_MODEL = 128
CHUNK = 128


@functools.cache
def _make_gather(n_total_chunks):
    info = plsc.get_sparse_core_info()
    nc, ns = info.num_cores, info.num_subcores
    nw = nc * ns
    chunks_per_w = n_total_chunks // nw
    mesh = plsc.VectorSubcoreMesh(core_axis_name="c", subcore_axis_name="s")

    def body(x_hbm, w_hbm, out_hbm, idx_v, rows_v, sem):
        wid = lax.axis_index("s") * nc + lax.axis_index("c")
        base_chunk = wid * chunks_per_w
        pltpu.sync_copy(x_hbm.at[pl.ds(base_chunk, chunks_per_w)], idx_v)

        def step(j, carry):
            pltpu.async_copy(w_hbm.at[idx_v.at[j]], rows_v, sem).wait()
            pltpu.sync_copy(
                rows_v, out_hbm.at[pl.ds((base_chunk + j) * CHUNK, CHUNK)]
            )
            return carry

        lax.fori_loop(0, chunks_per_w, step, 0)

    return pl.kernel(
        body,
        out_type=jax.ShapeDtypeStruct((n_total_chunks * CHUNK, D_MODEL), jnp.float32),
        mesh=mesh,
        scratch_types=[
            pltpu.VMEM((chunks_per_w, CHUNK), jnp.int32),
            pltpu.VMEM((CHUNK, D_MODEL), jnp.float32),
            pltpu.SemaphoreType.DMA,
        ],
    )


def kernel(x, W):
    b, t = x.shape
    n = b * t
    x2d = x.reshape(n // CHUNK, CHUNK).astype(jnp.int32)
    out = _make_gather(n // CHUNK)(x2d, W)
    return out.reshape(b, t, D_MODEL)


# speedup vs baseline: 9.1974x; 1.4541x over previous
import functools

import jax
import jax.numpy as jnp
from jax import lax
from jax.experimental import pallas as pl
from jax.experimental.pallas import tpu as pltpu
from jax.experimental.pallas import tpu_sc as plsc

D_MODEL = 128
CHUNK = 128
NBUF = 5


@functools.cache
def _make_gather(n_total_chunks):
    info = plsc.get_sparse_core_info()
    nc, ns = info.num_cores, info.num_subcores
    nw = nc * ns
    chunks_per_w = n_total_chunks // nw
    n_groups = chunks_per_w // NBUF
    mesh = plsc.VectorSubcoreMesh(core_axis_name="c", subcore_axis_name="s")

    def body(x_hbm, w_hbm, out_hbm, idx_v, rows_v, gsem, wsem):
        wid = lax.axis_index("s") * nc + lax.axis_index("c")
        base_chunk = wid * chunks_per_w
        pltpu.sync_copy(x_hbm.at[pl.ds(base_chunk, chunks_per_w)], idx_v)

        def start_gather(j, b):
            pltpu.async_copy(w_hbm.at[idx_v.at[j]], rows_v.at[b], gsem)

        def start_write(j, b):
            pltpu.async_copy(
                rows_v.at[b], out_hbm.at[pl.ds((base_chunk + j) * CHUNK, CHUNK)], wsem
            )

        def wait_gather(b):
            pltpu.make_async_copy(w_hbm.at[idx_v.at[0]], rows_v.at[b], gsem).wait()

        def wait_write(j, b):
            pltpu.make_async_copy(
                rows_v.at[b], out_hbm.at[pl.ds((base_chunk + j) * CHUNK, CHUNK)], wsem
            ).wait()

        for b in range(NBUF):
            start_gather(b, b)

        def group(g, carry):
            for b in range(NBUF):
                j = g * NBUF + b
                wait_gather(b)
                start_write(j, b)
                wait_write(j, b)
                start_gather(j + NBUF, b)
            return carry

        lax.fori_loop(0, n_groups - 1, group, 0)

        for b in range(NBUF):
            j = (n_groups - 1) * NBUF + b
            wait_gather(b)
            start_write(j, b)
            wait_write(j, b)

    return pl.kernel(
        body,
        out_type=jax.ShapeDtypeStruct((n_total_chunks * CHUNK, D_MODEL), jnp.float32),
        mesh=mesh,
        scratch_types=[
            pltpu.VMEM((chunks_per_w, CHUNK), jnp.int32),
            pltpu.VMEM((NBUF, CHUNK, D_MODEL), jnp.float32),
            pltpu.SemaphoreType.DMA,
            pltpu.SemaphoreType.DMA,
        ],
    )


def kernel(x, W):
    b, t = x.shape
    n = b * t
    x2d = x.reshape(n // CHUNK, CHUNK).astype(jnp.int32)
    out = _make_gather(n // CHUNK)(x2d, W)
    return out.reshape(b, t, D_MODEL)
